# baseline (speedup 1.0000x reference)
import jax
import jax.numpy as jnp
from jax import lax
from jax.experimental import pallas as pl
from jax.experimental.pallas import tpu as pltpu
from jax.experimental.pallas import tpu_sc as plsc

NODE_DIM = 128
EDGE_DIM = 16
N_NODES = 10000
N_EDGES = 320000

NC, NS = 2, 16
NW = NC * NS
E_PER_W = N_EDGES // NW
CHUNK = 48
N_CHUNKS = E_PER_W // CHUNK
TAIL = E_PER_W - N_CHUNKS * CHUNK
N_NODES_PAD = 10240
ROWS_PER_TILE = N_NODES_PAD // NS
ZROWS = 128

_f32 = jnp.float32


def _node_proj_body(x_ref, ws_ref, wt_ref, xs_ref, xt_ref):
    x = x_ref[...]
    xs_ref[...] = jnp.dot(x, ws_ref[...], preferred_element_type=_f32)
    xt_ref[...] = jnp.dot(x, wt_ref[...], preferred_element_type=_f32)


def _edge_proj_body(attr_ref, we_ref, be_ref, ea_ref):
    ea = (
        jnp.dot(attr_ref[...], we_ref[...], preferred_element_type=_f32)
        + be_ref[...]
    )
    eb = jax.lax.bitcast_convert_type(
        ea.astype(jnp.bfloat16), jnp.uint16
    ).astype(jnp.uint32)
    pairs = eb.reshape(eb.shape[0] // 2, 2, NODE_DIM)
    packed = pairs[:, 0, :] | (pairs[:, 1, :] << 16)
    ea_ref[...] = jax.lax.bitcast_convert_type(packed, jnp.int32)


def _sc_edge_body(xs_hbm, xt_hbm, ea_hbm, row_hbm, col_hbm,
                  acc_hbm, deg_hbm,
                  row0, col0, row1, col1, rowt, colt,
                  ea0, xt0, h0, ea1, xt1, h1,
                  deg_v, tmp_a, tmp_b, acc_sh,
                  sx0, st0, se0, ss0, sx1, st1, se1, ss1):
    c = lax.axis_index("c")
    s = lax.axis_index("s")
    wid = c * NS + s

    zvec = jnp.zeros((16,), _f32)

    def dzero(i, _):
        deg_v[pl.ds(i * 16, 16)] = zvec
        return 0

    lax.fori_loop(0, N_NODES_PAD // 16, dzero, 0)

    def zrow(i, _):
        for j in range(NODE_DIM // 16):
            xt0[i, pl.ds(j * 16, 16)] = zvec
        return 0

    lax.fori_loop(0, CHUNK, zrow, 0)

    def zcopy(i, _):
        pltpu.sync_copy(
            xt0, acc_sh.at[pl.ds(s * ROWS_PER_TILE + i * CHUNK, CHUNK)]
        )
        return 0

    lax.fori_loop(0, ROWS_PER_TILE // CHUNK, zcopy, 0)

    pltpu.sync_copy(
        xt0.at[pl.ds(0, 16)],
        acc_sh.at[pl.ds(s * ROWS_PER_TILE + (ROWS_PER_TILE // CHUNK) * CHUNK, 16)],
    )

    plsc.subcore_barrier()

    lane = lax.broadcasted_iota(jnp.int32, (16,), 0)

    def count_degrees(idx):
        srt, _ = plsc.sort_key_val(idx, idx)
        tmp_a[pl.ds(0, 16)] = srt
        nxt = plsc.load_gather(tmp_a, [jnp.minimum(lane + 1, 15)])
        is_last = jnp.logical_or(srt != nxt, lane == 15)
        cm = plsc.cummax(jnp.where(is_last, lane, -1))
        tmp_b[pl.ds(0, 16)] = cm
        prev = plsc.load_gather(tmp_b, [jnp.maximum(lane - 1, 0)])
        prev = jnp.where(lane == 0, -1, prev)
        cnt = (lane - prev).astype(_f32)
        plsc.addupdate_scatter(deg_v, [srt], cnt, mask=is_last)

    himask = jnp.broadcast_to(jnp.uint32(0xFFFF0000), (16,))

    def relu_rows(ea_v, xt_v, h_v, npairs):
        def pairbody(p, _):
            r0 = 2 * p
            r1 = 2 * p + 1
            for j in range(NODE_DIM // 16):
                sl = pl.ds(j * 16, 16)
                u = plsc.bitcast(ea_v[p, sl], jnp.uint32)
                elo = plsc.bitcast(u << 16, _f32)
                ehi = plsc.bitcast(u & himask, _f32)
                h_v[r0, sl] = jnp.maximum(h_v[r0, sl] + xt_v[r0, sl] + elo, 0.0)
                h_v[r1, sl] = jnp.maximum(h_v[r1, sl] + xt_v[r1, sl] + ehi, 0.0)
            return 0

        lax.fori_loop(0, npairs, pairbody, 0)

    base_t = wid * E_PER_W + N_CHUNKS * CHUNK
    pltpu.sync_copy(row_hbm.at[pl.ds(base_t, TAIL)], rowt)
    pltpu.sync_copy(col_hbm.at[pl.ds(base_t, TAIL)], colt)
    g1 = pltpu.async_copy(xs_hbm.at[rowt], h0.at[pl.ds(0, TAIL)], sx0)
    g2 = pltpu.async_copy(xt_hbm.at[colt], xt0.at[pl.ds(0, TAIL)], st0)
    pltpu.sync_copy(ea_hbm.at[pl.ds(pl.multiple_of(base_t // 2, 8), TAIL // 2)],
                    ea0.at[pl.ds(0, TAIL // 2)])
    g1.wait()
    g2.wait()
    relu_rows(ea0, xt0, h0, TAIL // 2)
    count_degrees(colt[pl.ds(0, 16)])
    pltpu.sync_copy(h0.at[pl.ds(0, TAIL)], acc_sh.at[colt], add=True)

    bufs = ((row0, col0, ea0, xt0, h0, sx0, st0, se0, ss0),
            (row1, col1, ea1, xt1, h1, sx1, st1, se1, ss1))

    def prefetch(b, t):
        (r, cl, ea_v, xt_v, h_v, se_x, se_t, se_e, _) = b
        base = wid * E_PER_W + t * CHUNK
        pltpu.sync_copy(row_hbm.at[pl.ds(base, CHUNK)], r)
        pltpu.sync_copy(col_hbm.at[pl.ds(base, CHUNK)], cl)
        pltpu.async_copy(xs_hbm.at[r], h_v, se_x)
        pltpu.async_copy(xt_hbm.at[cl], xt_v, se_t)
        pltpu.async_copy(ea_hbm.at[pl.ds(pl.multiple_of(base // 2, 8), CHUNK // 2)], ea_v, se_e)

    def wait_in(b, t):
        (r, cl, ea_v, xt_v, h_v, se_x, se_t, se_e, _) = b
        base = wid * E_PER_W + t * CHUNK
        pltpu.make_async_copy(xs_hbm.at[r], h_v, se_x).wait()
        pltpu.make_async_copy(xt_hbm.at[cl], xt_v, se_t).wait()
        pltpu.make_async_copy(
            ea_hbm.at[pl.ds(pl.multiple_of(base // 2, 8), CHUNK // 2)],
            ea_v, se_e).wait()

    def scatter_issue(b):
        (_, cl, _, _, h_v, _, _, _, se_s) = b
        pltpu.async_copy(h_v, acc_sh.at[cl], se_s, add=True)

    def scatter_drain(b):
        (_, cl, _, _, h_v, _, _, _, se_s) = b
        pltpu.make_async_copy(h_v, acc_sh.at[cl], se_s).wait()

    def count_chunk(b):
        cl = b[1]
        for g in range(CHUNK // 16):
            count_degrees(cl[pl.ds(g * 16, 16)])

    def process(b):
        (_, _, ea_v, xt_v, h_v, _, _, _, _) = b
        relu_rows(ea_v, xt_v, h_v, CHUNK // 2)
        scatter_issue(b)
        count_chunk(b)

    prefetch(bufs[0], 0)
    prefetch(bufs[1], 1)

    def pair(k, _):
        t0 = 2 * k
        wait_in(bufs[0], t0)
        process(bufs[0])
        wait_in(bufs[1], t0 + 1)
        scatter_drain(bufs[0])
        prefetch(bufs[0], t0 + 2)
        process(bufs[1])
        scatter_drain(bufs[1])
        prefetch(bufs[1], t0 + 3)
        return 0

    lax.fori_loop(0, (N_CHUNKS - 2) // 2, pair, 0)

    wait_in(bufs[0], N_CHUNKS - 2)
    process(bufs[0])
    wait_in(bufs[1], N_CHUNKS - 1)
    process(bufs[1])
    scatter_drain(bufs[0])
    scatter_drain(bufs[1])

    plsc.subcore_barrier()

    r0 = s * ROWS_PER_TILE
    pltpu.sync_copy(
        acc_sh.at[pl.ds(r0, ROWS_PER_TILE)],
        acc_hbm.at[pl.ds(c * N_NODES_PAD + r0, ROWS_PER_TILE)],
    )
    pltpu.sync_copy(deg_v, deg_hbm.at[wid])


def _phase_b(xs, xt, ea, row, col):
    mesh = plsc.VectorSubcoreMesh(core_axis_name="c", subcore_axis_name="s")
    idx_t = lambda n: pltpu.VMEM((n,), jnp.int32)
    buf_t = lambda n: pltpu.VMEM((n, NODE_DIM), _f32)
    return pl.kernel(
        _sc_edge_body,
        out_type=(
            jax.ShapeDtypeStruct((NC * N_NODES_PAD, NODE_DIM), _f32),
            jax.ShapeDtypeStruct((NW, N_NODES_PAD), _f32),
        ),
        mesh=mesh,
        compiler_params=pltpu.CompilerParams(needs_layout_passes=False),
        scratch_types=[
            idx_t(CHUNK), idx_t(CHUNK), idx_t(CHUNK), idx_t(CHUNK),
            idx_t(TAIL), idx_t(TAIL),
            pltpu.VMEM((CHUNK // 2, NODE_DIM), jnp.int32), buf_t(CHUNK), buf_t(CHUNK),
            pltpu.VMEM((CHUNK // 2, NODE_DIM), jnp.int32), buf_t(CHUNK), buf_t(CHUNK),
            pltpu.VMEM((N_NODES_PAD,), _f32),
            pltpu.VMEM((16,), jnp.int32),
            pltpu.VMEM((16,), jnp.int32),
            pltpu.VMEM_SHARED((N_NODES_PAD, NODE_DIM), _f32),
            pltpu.SemaphoreType.DMA, pltpu.SemaphoreType.DMA,
            pltpu.SemaphoreType.DMA, pltpu.SemaphoreType.DMA,
            pltpu.SemaphoreType.DMA, pltpu.SemaphoreType.DMA,
            pltpu.SemaphoreType.DMA, pltpu.SemaphoreType.DMA,
        ],
    )(xs, xt, ea, row, col)


def _node_mlp_body(agg0_ref, agg1_ref, degt_ref, x_ref, we2_ref, be2_ref,
                   wn1a_ref, wn1x_ref, bn1_ref, wn2_ref, bn2_ref, out_ref):
    aggsum = agg0_ref[...] + agg1_ref[...]
    deg = jnp.sum(degt_ref[...], axis=1, keepdims=True)
    aggregated = (
        jnp.dot(aggsum, we2_ref[...], preferred_element_type=_f32)
        + deg * be2_ref[...]
    )
    x = x_ref[...]
    h2 = jnp.maximum(
        jnp.dot(x, wn1x_ref[...], preferred_element_type=_f32)
        + jnp.dot(aggregated, wn1a_ref[...], preferred_element_type=_f32)
        + bn1_ref[...],
        0.0,
    )
    out_ref[...] = jnp.maximum(
        jnp.dot(h2, wn2_ref[...], preferred_element_type=_f32)
        + bn2_ref[...] + x,
        0.0,
    )


def _phase_c(agg0, agg1, degt, x, we2, be2, wn1a, wn1x, bn1, wn2, bn2):
    def full(r, c):
        return pl.BlockSpec((r, c), lambda i: (0, 0))

    return pl.pallas_call(
        _node_mlp_body,
        grid=(5,),
        in_specs=[
            pl.BlockSpec((2000, NODE_DIM), lambda i: (i, 0)),
            pl.BlockSpec((2000, NODE_DIM), lambda i: (i, 0)),
            pl.BlockSpec((2000, NW), lambda i: (i, 0)),
            pl.BlockSpec((2000, NODE_DIM), lambda i: (i, 0)),
            full(NODE_DIM, NODE_DIM),
            full(1, NODE_DIM),
            full(NODE_DIM, NODE_DIM),
            full(NODE_DIM, NODE_DIM),
            full(1, NODE_DIM),
            full(NODE_DIM, NODE_DIM),
            full(1, NODE_DIM),
        ],
        out_specs=pl.BlockSpec((2000, NODE_DIM), lambda i: (i, 0)),
        out_shape=jax.ShapeDtypeStruct((N_NODES, NODE_DIM), _f32),
    )(agg0, agg1, degt, x, we2, be2, wn1a, wn1x, bn1, wn2, bn2)


def kernel(x, edge_index, edge_attr, We1, be1, We2, be2, Wn1, bn1, Wn2, bn2):
    row = edge_index[0].astype(jnp.int32)
    col = edge_index[1].astype(jnp.int32)

    ws = We1[:NODE_DIM]
    wt = We1[NODE_DIM:2 * NODE_DIM]
    we = We1[2 * NODE_DIM:]

    xs, xt = pl.pallas_call(
        _node_proj_body,
        grid=(5,),
        in_specs=[
            pl.BlockSpec((2000, NODE_DIM), lambda i: (i, 0)),
            pl.BlockSpec((NODE_DIM, NODE_DIM), lambda i: (0, 0)),
            pl.BlockSpec((NODE_DIM, NODE_DIM), lambda i: (0, 0)),
        ],
        out_specs=[
            pl.BlockSpec((2000, NODE_DIM), lambda i: (i, 0)),
            pl.BlockSpec((2000, NODE_DIM), lambda i: (i, 0)),
        ],
        out_shape=[jax.ShapeDtypeStruct((N_NODES, NODE_DIM), _f32)] * 2,
    )(x, ws, wt)

    ea = pl.pallas_call(
        _edge_proj_body,
        grid=(20,),
        in_specs=[
            pl.BlockSpec((16000, EDGE_DIM), lambda i: (i, 0)),
            pl.BlockSpec((EDGE_DIM, NODE_DIM), lambda i: (0, 0)),
            pl.BlockSpec((1, NODE_DIM), lambda i: (0, 0)),
        ],
        out_specs=pl.BlockSpec((8000, NODE_DIM), lambda i: (i, 0)),
        out_shape=jax.ShapeDtypeStruct((N_EDGES // 2, NODE_DIM), jnp.int32),
    )(edge_attr, we, be1.reshape(1, NODE_DIM))

    acc, deg = _phase_b(xs, xt, ea, row, col)

    degt = deg.T[:N_NODES]
    return _phase_c(
        acc[:N_NODES], acc[N_NODES_PAD:N_NODES_PAD + N_NODES], degt, x,
        We2, be2.reshape(1, NODE_DIM),
        Wn1[NODE_DIM:], Wn1[:NODE_DIM],
        bn1.reshape(1, NODE_DIM), Wn2, bn2.reshape(1, NODE_DIM),
    )


# speedup vs baseline: 1.0653x; 1.0653x over previous
import jax
import jax.numpy as jnp
from jax import lax
from jax.experimental import pallas as pl
from jax.experimental.pallas import tpu as pltpu
from jax.experimental.pallas import tpu_sc as plsc

NODE_DIM = 128
EDGE_DIM = 16
N_NODES = 10000
N_EDGES = 320000

NC, NS = 2, 16
NW = NC * NS
E_PER_W = N_EDGES // NW
CHUNK = 48
N_CHUNKS = E_PER_W // CHUNK
TAIL = E_PER_W - N_CHUNKS * CHUNK
N_NODES_PAD = 10240
ROWS_PER_TILE = N_NODES_PAD // NS
ZROWS = 128

_f32 = jnp.float32


def _node_proj_body(x_ref, ws_ref, wt_ref, xs_ref, xt_ref):
    x = x_ref[...]
    xs_ref[...] = jnp.dot(x, ws_ref[...], preferred_element_type=_f32)
    xt_ref[...] = jnp.dot(x, wt_ref[...], preferred_element_type=_f32)


def _edge_proj_body(attr2_ref, we_ref, be_ref, ea_ref):
    attr2 = attr2_ref[...]
    lo = (
        jnp.dot(attr2[:, :EDGE_DIM], we_ref[...], preferred_element_type=_f32)
        + be_ref[...]
    )
    hi = (
        jnp.dot(attr2[:, EDGE_DIM:], we_ref[...], preferred_element_type=_f32)
        + be_ref[...]
    )
    lou = jax.lax.bitcast_convert_type(
        lo.astype(jnp.bfloat16), jnp.uint16).astype(jnp.uint32)
    hiu = jax.lax.bitcast_convert_type(
        hi.astype(jnp.bfloat16), jnp.uint16).astype(jnp.uint32)
    ea_ref[...] = jax.lax.bitcast_convert_type(lou | (hiu << 16), jnp.int32)


def _sc_edge_body(xs_hbm, xt_hbm, ea_hbm, row_hbm, col_hbm,
                  acc_hbm, deg_hbm,
                  row0, col0, row1, col1, rowt, colt,
                  ea0, xt0, h0, ea1, xt1, h1,
                  deg_v, tmp_a, tmp_b, acc_sh,
                  sx0, st0, se0, ss0, sx1, st1, se1, ss1):
    c = lax.axis_index("c")
    s = lax.axis_index("s")
    wid = c * NS + s

    zvec = jnp.zeros((16,), _f32)

    def dzero(i, _):
        deg_v[pl.ds(i * 16, 16)] = zvec
        return 0

    lax.fori_loop(0, N_NODES_PAD // 16, dzero, 0)

    def zrow(i, _):
        for j in range(NODE_DIM // 16):
            xt0[i, pl.ds(j * 16, 16)] = zvec
        return 0

    lax.fori_loop(0, CHUNK, zrow, 0)

    def zcopy(i, _):
        pltpu.sync_copy(
            xt0, acc_sh.at[pl.ds(s * ROWS_PER_TILE + i * CHUNK, CHUNK)]
        )
        return 0

    lax.fori_loop(0, ROWS_PER_TILE // CHUNK, zcopy, 0)

    pltpu.sync_copy(
        xt0.at[pl.ds(0, 16)],
        acc_sh.at[pl.ds(s * ROWS_PER_TILE + (ROWS_PER_TILE // CHUNK) * CHUNK, 16)],
    )

    plsc.subcore_barrier()

    lane = lax.broadcasted_iota(jnp.int32, (16,), 0)

    def count_degrees(idx):
        srt, _ = plsc.sort_key_val(idx, idx)
        tmp_a[pl.ds(0, 16)] = srt
        nxt = plsc.load_gather(tmp_a, [jnp.minimum(lane + 1, 15)])
        is_last = jnp.logical_or(srt != nxt, lane == 15)
        cm = plsc.cummax(jnp.where(is_last, lane, -1))
        tmp_b[pl.ds(0, 16)] = cm
        prev = plsc.load_gather(tmp_b, [jnp.maximum(lane - 1, 0)])
        prev = jnp.where(lane == 0, -1, prev)
        cnt = (lane - prev).astype(_f32)
        plsc.addupdate_scatter(deg_v, [srt], cnt, mask=is_last)

    himask = jnp.broadcast_to(jnp.uint32(0xFFFF0000), (16,))

    def relu_rows(ea_v, xt_v, h_v, npairs):
        def pairbody(p, _):
            r0 = 2 * p
            r1 = 2 * p + 1
            for j in range(NODE_DIM // 16):
                sl = pl.ds(j * 16, 16)
                u = plsc.bitcast(ea_v[p, sl], jnp.uint32)
                elo = plsc.bitcast(u << 16, _f32)
                ehi = plsc.bitcast(u & himask, _f32)
                h_v[r0, sl] = jnp.maximum(h_v[r0, sl] + xt_v[r0, sl] + elo, 0.0)
                h_v[r1, sl] = jnp.maximum(h_v[r1, sl] + xt_v[r1, sl] + ehi, 0.0)
            return 0

        lax.fori_loop(0, npairs, pairbody, 0)

    base_t = wid * E_PER_W + N_CHUNKS * CHUNK
    pltpu.sync_copy(row_hbm.at[pl.ds(base_t, TAIL)], rowt)
    pltpu.sync_copy(col_hbm.at[pl.ds(base_t, TAIL)], colt)
    g1 = pltpu.async_copy(xs_hbm.at[rowt], h0.at[pl.ds(0, TAIL)], sx0)
    g2 = pltpu.async_copy(xt_hbm.at[colt], xt0.at[pl.ds(0, TAIL)], st0)
    pltpu.sync_copy(ea_hbm.at[pl.ds(pl.multiple_of(base_t // 2, 8), TAIL // 2)],
                    ea0.at[pl.ds(0, TAIL // 2)])
    g1.wait()
    g2.wait()
    relu_rows(ea0, xt0, h0, TAIL // 2)
    count_degrees(colt[pl.ds(0, 16)])
    pltpu.sync_copy(h0.at[pl.ds(0, TAIL)], acc_sh.at[colt], add=True)

    bufs = ((row0, col0, ea0, xt0, h0, sx0, st0, se0, ss0),
            (row1, col1, ea1, xt1, h1, sx1, st1, se1, ss1))

    def prefetch(b, t):
        (r, cl, ea_v, xt_v, h_v, se_x, se_t, se_e, _) = b
        base = wid * E_PER_W + t * CHUNK
        pltpu.sync_copy(row_hbm.at[pl.ds(base, CHUNK)], r)
        pltpu.sync_copy(col_hbm.at[pl.ds(base, CHUNK)], cl)
        pltpu.async_copy(xs_hbm.at[r], h_v, se_x)
        pltpu.async_copy(xt_hbm.at[cl], xt_v, se_t)
        pltpu.async_copy(ea_hbm.at[pl.ds(pl.multiple_of(base // 2, 8), CHUNK // 2)], ea_v, se_e)

    def wait_in(b, t):
        (r, cl, ea_v, xt_v, h_v, se_x, se_t, se_e, _) = b
        base = wid * E_PER_W + t * CHUNK
        pltpu.make_async_copy(xs_hbm.at[r], h_v, se_x).wait()
        pltpu.make_async_copy(xt_hbm.at[cl], xt_v, se_t).wait()
        pltpu.make_async_copy(
            ea_hbm.at[pl.ds(pl.multiple_of(base // 2, 8), CHUNK // 2)],
            ea_v, se_e).wait()

    def scatter_issue(b):
        (_, cl, _, _, h_v, _, _, _, se_s) = b
        pltpu.async_copy(h_v, acc_sh.at[cl], se_s, add=True)

    def scatter_drain(b):
        (_, cl, _, _, h_v, _, _, _, se_s) = b
        pltpu.make_async_copy(h_v, acc_sh.at[cl], se_s).wait()

    def count_chunk(b):
        cl = b[1]
        for g in range(CHUNK // 16):
            count_degrees(cl[pl.ds(g * 16, 16)])

    def process(b):
        (_, _, ea_v, xt_v, h_v, _, _, _, _) = b
        relu_rows(ea_v, xt_v, h_v, CHUNK // 2)
        scatter_issue(b)
        count_chunk(b)

    prefetch(bufs[0], 0)
    prefetch(bufs[1], 1)

    def pair(k, _):
        t0 = 2 * k
        wait_in(bufs[0], t0)
        process(bufs[0])
        wait_in(bufs[1], t0 + 1)
        scatter_drain(bufs[0])
        prefetch(bufs[0], t0 + 2)
        process(bufs[1])
        scatter_drain(bufs[1])
        prefetch(bufs[1], t0 + 3)
        return 0

    lax.fori_loop(0, (N_CHUNKS - 2) // 2, pair, 0)

    wait_in(bufs[0], N_CHUNKS - 2)
    process(bufs[0])
    wait_in(bufs[1], N_CHUNKS - 1)
    process(bufs[1])
    scatter_drain(bufs[0])
    scatter_drain(bufs[1])

    plsc.subcore_barrier()

    r0 = s * ROWS_PER_TILE
    pltpu.sync_copy(
        acc_sh.at[pl.ds(r0, ROWS_PER_TILE)],
        acc_hbm.at[pl.ds(c * N_NODES_PAD + r0, ROWS_PER_TILE)],
    )
    pltpu.sync_copy(deg_v, deg_hbm.at[wid])


def _phase_b(xs, xt, ea, row, col):
    mesh = plsc.VectorSubcoreMesh(core_axis_name="c", subcore_axis_name="s")
    idx_t = lambda n: pltpu.VMEM((n,), jnp.int32)
    buf_t = lambda n: pltpu.VMEM((n, NODE_DIM), _f32)
    return pl.kernel(
        _sc_edge_body,
        out_type=(
            jax.ShapeDtypeStruct((NC * N_NODES_PAD, NODE_DIM), _f32),
            jax.ShapeDtypeStruct((NW, N_NODES_PAD), _f32),
        ),
        mesh=mesh,
        compiler_params=pltpu.CompilerParams(needs_layout_passes=False),
        scratch_types=[
            idx_t(CHUNK), idx_t(CHUNK), idx_t(CHUNK), idx_t(CHUNK),
            idx_t(TAIL), idx_t(TAIL),
            pltpu.VMEM((CHUNK // 2, NODE_DIM), jnp.int32), buf_t(CHUNK), buf_t(CHUNK),
            pltpu.VMEM((CHUNK // 2, NODE_DIM), jnp.int32), buf_t(CHUNK), buf_t(CHUNK),
            pltpu.VMEM((N_NODES_PAD,), _f32),
            pltpu.VMEM((16,), jnp.int32),
            pltpu.VMEM((16,), jnp.int32),
            pltpu.VMEM_SHARED((N_NODES_PAD, NODE_DIM), _f32),
            pltpu.SemaphoreType.DMA, pltpu.SemaphoreType.DMA,
            pltpu.SemaphoreType.DMA, pltpu.SemaphoreType.DMA,
            pltpu.SemaphoreType.DMA, pltpu.SemaphoreType.DMA,
            pltpu.SemaphoreType.DMA, pltpu.SemaphoreType.DMA,
        ],
    )(xs, xt, ea, row, col)


def _node_mlp_body(agg0_ref, agg1_ref, degt_ref, x_ref, we2_ref, be2_ref,
                   wn1a_ref, wn1x_ref, bn1_ref, wn2_ref, bn2_ref, out_ref):
    aggsum = agg0_ref[...] + agg1_ref[...]
    deg = jnp.sum(degt_ref[...], axis=1, keepdims=True)
    aggregated = (
        jnp.dot(aggsum, we2_ref[...], preferred_element_type=_f32)
        + deg * be2_ref[...]
    )
    x = x_ref[...]
    h2 = jnp.maximum(
        jnp.dot(x, wn1x_ref[...], preferred_element_type=_f32)
        + jnp.dot(aggregated, wn1a_ref[...], preferred_element_type=_f32)
        + bn1_ref[...],
        0.0,
    )
    out_ref[...] = jnp.maximum(
        jnp.dot(h2, wn2_ref[...], preferred_element_type=_f32)
        + bn2_ref[...] + x,
        0.0,
    )


def _phase_c(agg0, agg1, degt, x, we2, be2, wn1a, wn1x, bn1, wn2, bn2):
    def full(r, c):
        return pl.BlockSpec((r, c), lambda i: (0, 0))

    return pl.pallas_call(
        _node_mlp_body,
        grid=(5,),
        in_specs=[
            pl.BlockSpec((2000, NODE_DIM), lambda i: (i, 0)),
            pl.BlockSpec((2000, NODE_DIM), lambda i: (i, 0)),
            pl.BlockSpec((2000, NW), lambda i: (i, 0)),
            pl.BlockSpec((2000, NODE_DIM), lambda i: (i, 0)),
            full(NODE_DIM, NODE_DIM),
            full(1, NODE_DIM),
            full(NODE_DIM, NODE_DIM),
            full(NODE_DIM, NODE_DIM),
            full(1, NODE_DIM),
            full(NODE_DIM, NODE_DIM),
            full(1, NODE_DIM),
        ],
        out_specs=pl.BlockSpec((2000, NODE_DIM), lambda i: (i, 0)),
        out_shape=jax.ShapeDtypeStruct((N_NODES, NODE_DIM), _f32),
    )(agg0, agg1, degt, x, we2, be2, wn1a, wn1x, bn1, wn2, bn2)


def kernel(x, edge_index, edge_attr, We1, be1, We2, be2, Wn1, bn1, Wn2, bn2):
    row = edge_index[0].astype(jnp.int32)
    col = edge_index[1].astype(jnp.int32)

    ws = We1[:NODE_DIM]
    wt = We1[NODE_DIM:2 * NODE_DIM]
    we = We1[2 * NODE_DIM:]

    xs, xt = pl.pallas_call(
        _node_proj_body,
        grid=(5,),
        in_specs=[
            pl.BlockSpec((2000, NODE_DIM), lambda i: (i, 0)),
            pl.BlockSpec((NODE_DIM, NODE_DIM), lambda i: (0, 0)),
            pl.BlockSpec((NODE_DIM, NODE_DIM), lambda i: (0, 0)),
        ],
        out_specs=[
            pl.BlockSpec((2000, NODE_DIM), lambda i: (i, 0)),
            pl.BlockSpec((2000, NODE_DIM), lambda i: (i, 0)),
        ],
        out_shape=[jax.ShapeDtypeStruct((N_NODES, NODE_DIM), _f32)] * 2,
    )(x, ws, wt)

    ea = pl.pallas_call(
        _edge_proj_body,
        grid=(20,),
        in_specs=[
            pl.BlockSpec((8000, 2 * EDGE_DIM), lambda i: (i, 0)),
            pl.BlockSpec((EDGE_DIM, NODE_DIM), lambda i: (0, 0)),
            pl.BlockSpec((1, NODE_DIM), lambda i: (0, 0)),
        ],
        out_specs=pl.BlockSpec((8000, NODE_DIM), lambda i: (i, 0)),
        out_shape=jax.ShapeDtypeStruct((N_EDGES // 2, NODE_DIM), jnp.int32),
    )(edge_attr.reshape(N_EDGES // 2, 2 * EDGE_DIM), we, be1.reshape(1, NODE_DIM))

    acc, deg = _phase_b(xs, xt, ea, row, col)

    degt = deg.T[:N_NODES]
    return _phase_c(
        acc[:N_NODES], acc[N_NODES_PAD:N_NODES_PAD + N_NODES], degt, x,
        We2, be2.reshape(1, NODE_DIM),
        Wn1[NODE_DIM:], Wn1[:NODE_DIM],
        bn1.reshape(1, NODE_DIM), Wn2, bn2.reshape(1, NODE_DIM),
    )


# speedup vs baseline: 1.1744x; 1.1024x over previous
import jax
import jax.numpy as jnp
from jax import lax
from jax.experimental import pallas as pl
from jax.experimental.pallas import tpu as pltpu
from jax.experimental.pallas import tpu_sc as plsc

NODE_DIM = 128
EDGE_DIM = 16
N_NODES = 10000
N_EDGES = 320000

NC, NS = 2, 16
NW = NC * NS
E_PER_W = N_EDGES // NW
CHUNK = 48
N_CHUNKS = E_PER_W // CHUNK
TAIL = E_PER_W - N_CHUNKS * CHUNK
N_NODES_PAD = 10240
ROWS_PER_TILE = N_NODES_PAD // NS
ZROWS = 128

_f32 = jnp.float32


def _node_proj_body(x_ref, ws_ref, wt_ref, xs_ref, xt_ref):
    x = x_ref[...]
    xs_ref[...] = jnp.dot(x, ws_ref[...], preferred_element_type=_f32)
    xt_ref[...] = jnp.dot(x, wt_ref[...], preferred_element_type=_f32)


def _edge_proj_body(attr_ref, we_ref, be_ref, ea_ref):
    ea_ref[...] = (
        jnp.dot(attr_ref[...], we_ref[...], preferred_element_type=_f32)
        + be_ref[...]
    )


def _sc_edge_body(xs_hbm, xt_hbm, ea_hbm, row_hbm, col_hbm,
                  acc_hbm, deg_hbm,
                  row0, col0, row1, col1, rowt, colt,
                  ea0, xt0, h0, ea1, xt1, h1,
                  deg_v, tmp_a, tmp_b, acc_sh,
                  sx0, st0, se0, ss0, sx1, st1, se1, ss1):
    c = lax.axis_index("c")
    s = lax.axis_index("s")
    wid = c * NS + s

    zvec = jnp.zeros((16,), _f32)

    def dzero(i, _):
        deg_v[pl.ds(i * 16, 16)] = zvec
        return 0

    lax.fori_loop(0, N_NODES_PAD // 16, dzero, 0)

    def zrow(i, _):
        for j in range(NODE_DIM // 16):
            xt0[i, pl.ds(j * 16, 16)] = zvec
        return 0

    lax.fori_loop(0, CHUNK, zrow, 0)

    def zcopy(i, _):
        pltpu.sync_copy(
            xt0, acc_sh.at[pl.ds(s * ROWS_PER_TILE + i * CHUNK, CHUNK)]
        )
        return 0

    lax.fori_loop(0, ROWS_PER_TILE // CHUNK, zcopy, 0)

    pltpu.sync_copy(
        xt0.at[pl.ds(0, 16)],
        acc_sh.at[pl.ds(s * ROWS_PER_TILE + (ROWS_PER_TILE // CHUNK) * CHUNK, 16)],
    )

    plsc.subcore_barrier()

    lane = lax.broadcasted_iota(jnp.int32, (16,), 0)

    def count_degrees(idx):
        srt, _ = plsc.sort_key_val(idx, idx)
        tmp_a[pl.ds(0, 16)] = srt
        nxt = plsc.load_gather(tmp_a, [jnp.minimum(lane + 1, 15)])
        is_last = jnp.logical_or(srt != nxt, lane == 15)
        cm = plsc.cummax(jnp.where(is_last, lane, -1))
        tmp_b[pl.ds(0, 16)] = cm
        prev = plsc.load_gather(tmp_b, [jnp.maximum(lane - 1, 0)])
        prev = jnp.where(lane == 0, -1, prev)
        cnt = (lane - prev).astype(_f32)
        plsc.addupdate_scatter(deg_v, [srt], cnt, mask=is_last)

    def relu_rows(ea_v, xt_v, h_v, npairs):
        def pairbody(p, _):
            r0 = 2 * p
            r1 = 2 * p + 1
            for j in range(NODE_DIM // 16):
                sl = pl.ds(j * 16, 16)
                h_v[r0, sl] = jnp.maximum(
                    h_v[r0, sl] + xt_v[r0, sl] + ea_v[r0, sl], 0.0)
                h_v[r1, sl] = jnp.maximum(
                    h_v[r1, sl] + xt_v[r1, sl] + ea_v[r1, sl], 0.0)
            return 0

        lax.fori_loop(0, npairs, pairbody, 0)

    base_t = wid * E_PER_W + N_CHUNKS * CHUNK
    pltpu.sync_copy(row_hbm.at[pl.ds(base_t, TAIL)], rowt)
    pltpu.sync_copy(col_hbm.at[pl.ds(base_t, TAIL)], colt)
    g1 = pltpu.async_copy(xs_hbm.at[rowt], h0.at[pl.ds(0, TAIL)], sx0)
    g2 = pltpu.async_copy(xt_hbm.at[colt], xt0.at[pl.ds(0, TAIL)], st0)
    pltpu.sync_copy(ea_hbm.at[pl.ds(base_t, TAIL)], ea0.at[pl.ds(0, TAIL)])
    g1.wait()
    g2.wait()
    relu_rows(ea0, xt0, h0, TAIL // 2)
    count_degrees(colt[pl.ds(0, 16)])
    pltpu.sync_copy(h0.at[pl.ds(0, TAIL)], acc_sh.at[colt], add=True)

    bufs = ((row0, col0, ea0, xt0, h0, sx0, st0, se0, ss0),
            (row1, col1, ea1, xt1, h1, sx1, st1, se1, ss1))

    def prefetch(b, t):
        (r, cl, ea_v, xt_v, h_v, se_x, se_t, se_e, _) = b
        base = wid * E_PER_W + t * CHUNK
        pltpu.sync_copy(row_hbm.at[pl.ds(base, CHUNK)], r)
        pltpu.sync_copy(col_hbm.at[pl.ds(base, CHUNK)], cl)
        pltpu.async_copy(xs_hbm.at[r], h_v, se_x)
        pltpu.async_copy(xt_hbm.at[cl], xt_v, se_t)
        pltpu.async_copy(ea_hbm.at[pl.ds(base, CHUNK)], ea_v, se_e)

    def wait_in(b, t):
        (r, cl, ea_v, xt_v, h_v, se_x, se_t, se_e, _) = b
        base = wid * E_PER_W + t * CHUNK
        pltpu.make_async_copy(xs_hbm.at[r], h_v, se_x).wait()
        pltpu.make_async_copy(xt_hbm.at[cl], xt_v, se_t).wait()
        pltpu.make_async_copy(ea_hbm.at[pl.ds(base, CHUNK)], ea_v, se_e).wait()

    def scatter_issue(b):
        (_, cl, _, _, h_v, _, _, _, se_s) = b
        pltpu.async_copy(h_v, acc_sh.at[cl], se_s, add=True)

    def scatter_drain(b):
        (_, cl, _, _, h_v, _, _, _, se_s) = b
        pltpu.make_async_copy(h_v, acc_sh.at[cl], se_s).wait()

    def count_chunk(b):
        cl = b[1]
        for g in range(CHUNK // 16):
            count_degrees(cl[pl.ds(g * 16, 16)])

    def process(b):
        (_, _, ea_v, xt_v, h_v, _, _, _, _) = b
        relu_rows(ea_v, xt_v, h_v, CHUNK // 2)
        scatter_issue(b)
        count_chunk(b)

    prefetch(bufs[0], 0)
    prefetch(bufs[1], 1)

    def pair(k, _):
        t0 = 2 * k
        wait_in(bufs[0], t0)
        process(bufs[0])
        wait_in(bufs[1], t0 + 1)
        scatter_drain(bufs[0])
        prefetch(bufs[0], t0 + 2)
        process(bufs[1])
        scatter_drain(bufs[1])
        prefetch(bufs[1], t0 + 3)
        return 0

    lax.fori_loop(0, (N_CHUNKS - 2) // 2, pair, 0)

    wait_in(bufs[0], N_CHUNKS - 2)
    process(bufs[0])
    wait_in(bufs[1], N_CHUNKS - 1)
    process(bufs[1])
    scatter_drain(bufs[0])
    scatter_drain(bufs[1])

    plsc.subcore_barrier()

    r0 = s * ROWS_PER_TILE
    pltpu.sync_copy(
        acc_sh.at[pl.ds(r0, ROWS_PER_TILE)],
        acc_hbm.at[pl.ds(c * N_NODES_PAD + r0, ROWS_PER_TILE)],
    )
    pltpu.sync_copy(deg_v, deg_hbm.at[wid])


def _phase_b(xs, xt, ea, row, col):
    mesh = plsc.VectorSubcoreMesh(core_axis_name="c", subcore_axis_name="s")
    idx_t = lambda n: pltpu.VMEM((n,), jnp.int32)
    buf_t = lambda n: pltpu.VMEM((n, NODE_DIM), _f32)
    return pl.kernel(
        _sc_edge_body,
        out_type=(
            jax.ShapeDtypeStruct((NC * N_NODES_PAD, NODE_DIM), _f32),
            jax.ShapeDtypeStruct((NW, N_NODES_PAD), _f32),
        ),
        mesh=mesh,
        compiler_params=pltpu.CompilerParams(needs_layout_passes=False),
        scratch_types=[
            idx_t(CHUNK), idx_t(CHUNK), idx_t(CHUNK), idx_t(CHUNK),
            idx_t(TAIL), idx_t(TAIL),
            buf_t(CHUNK), buf_t(CHUNK), buf_t(CHUNK),
            buf_t(CHUNK), buf_t(CHUNK), buf_t(CHUNK),
            pltpu.VMEM((N_NODES_PAD,), _f32),
            pltpu.VMEM((16,), jnp.int32),
            pltpu.VMEM((16,), jnp.int32),
            pltpu.VMEM_SHARED((N_NODES_PAD, NODE_DIM), _f32),
            pltpu.SemaphoreType.DMA, pltpu.SemaphoreType.DMA,
            pltpu.SemaphoreType.DMA, pltpu.SemaphoreType.DMA,
            pltpu.SemaphoreType.DMA, pltpu.SemaphoreType.DMA,
            pltpu.SemaphoreType.DMA, pltpu.SemaphoreType.DMA,
        ],
    )(xs, xt, ea, row, col)


def _node_mlp_body(agg0_ref, agg1_ref, degt_ref, x_ref, we2_ref, be2_ref,
                   wn1a_ref, wn1x_ref, bn1_ref, wn2_ref, bn2_ref, out_ref):
    aggsum = agg0_ref[...] + agg1_ref[...]
    deg = jnp.sum(degt_ref[...], axis=1, keepdims=True)
    aggregated = (
        jnp.dot(aggsum, we2_ref[...], preferred_element_type=_f32)
        + deg * be2_ref[...]
    )
    x = x_ref[...]
    h2 = jnp.maximum(
        jnp.dot(x, wn1x_ref[...], preferred_element_type=_f32)
        + jnp.dot(aggregated, wn1a_ref[...], preferred_element_type=_f32)
        + bn1_ref[...],
        0.0,
    )
    out_ref[...] = jnp.maximum(
        jnp.dot(h2, wn2_ref[...], preferred_element_type=_f32)
        + bn2_ref[...] + x,
        0.0,
    )


def _phase_c(agg0, agg1, degt, x, we2, be2, wn1a, wn1x, bn1, wn2, bn2):
    def full(r, c):
        return pl.BlockSpec((r, c), lambda i: (0, 0))

    return pl.pallas_call(
        _node_mlp_body,
        grid=(5,),
        in_specs=[
            pl.BlockSpec((2000, NODE_DIM), lambda i: (i, 0)),
            pl.BlockSpec((2000, NODE_DIM), lambda i: (i, 0)),
            pl.BlockSpec((2000, NW), lambda i: (i, 0)),
            pl.BlockSpec((2000, NODE_DIM), lambda i: (i, 0)),
            full(NODE_DIM, NODE_DIM),
            full(1, NODE_DIM),
            full(NODE_DIM, NODE_DIM),
            full(NODE_DIM, NODE_DIM),
            full(1, NODE_DIM),
            full(NODE_DIM, NODE_DIM),
            full(1, NODE_DIM),
        ],
        out_specs=pl.BlockSpec((2000, NODE_DIM), lambda i: (i, 0)),
        out_shape=jax.ShapeDtypeStruct((N_NODES, NODE_DIM), _f32),
    )(agg0, agg1, degt, x, we2, be2, wn1a, wn1x, bn1, wn2, bn2)


def kernel(x, edge_index, edge_attr, We1, be1, We2, be2, Wn1, bn1, Wn2, bn2):
    row = edge_index[0].astype(jnp.int32)
    col = edge_index[1].astype(jnp.int32)

    ws = We1[:NODE_DIM]
    wt = We1[NODE_DIM:2 * NODE_DIM]
    we = We1[2 * NODE_DIM:]

    xs, xt = pl.pallas_call(
        _node_proj_body,
        grid=(5,),
        in_specs=[
            pl.BlockSpec((2000, NODE_DIM), lambda i: (i, 0)),
            pl.BlockSpec((NODE_DIM, NODE_DIM), lambda i: (0, 0)),
            pl.BlockSpec((NODE_DIM, NODE_DIM), lambda i: (0, 0)),
        ],
        out_specs=[
            pl.BlockSpec((2000, NODE_DIM), lambda i: (i, 0)),
            pl.BlockSpec((2000, NODE_DIM), lambda i: (i, 0)),
        ],
        out_shape=[jax.ShapeDtypeStruct((N_NODES, NODE_DIM), _f32)] * 2,
    )(x, ws, wt)

    ea = pl.pallas_call(
        _edge_proj_body,
        grid=(20,),
        in_specs=[
            pl.BlockSpec((16000, EDGE_DIM), lambda i: (i, 0)),
            pl.BlockSpec((EDGE_DIM, NODE_DIM), lambda i: (0, 0)),
            pl.BlockSpec((1, NODE_DIM), lambda i: (0, 0)),
        ],
        out_specs=pl.BlockSpec((16000, NODE_DIM), lambda i: (i, 0)),
        out_shape=jax.ShapeDtypeStruct((N_EDGES, NODE_DIM), _f32),
    )(edge_attr, we, be1.reshape(1, NODE_DIM))

    acc, deg = _phase_b(xs, xt, ea, row, col)

    degt = deg.T[:N_NODES]
    return _phase_c(
        acc[:N_NODES], acc[N_NODES_PAD:N_NODES_PAD + N_NODES], degt, x,
        We2, be2.reshape(1, NODE_DIM),
        Wn1[NODE_DIM:], Wn1[:NODE_DIM],
        bn1.reshape(1, NODE_DIM), Wn2, bn2.reshape(1, NODE_DIM),
    )


# speedup vs baseline: 1.5390x; 1.3105x over previous
import jax
import jax.numpy as jnp
from jax import lax
from jax.experimental import pallas as pl
from jax.experimental.pallas import tpu as pltpu
from jax.experimental.pallas import tpu_sc as plsc

NODE_DIM = 128
EDGE_DIM = 16
N_NODES = 10000
N_EDGES = 320000

NC, NS = 2, 16
NW = NC * NS
E_PER_W = N_EDGES // NW
CHUNK = 48
N_CHUNKS = E_PER_W // CHUNK
TAIL = E_PER_W - N_CHUNKS * CHUNK
N_NODES_PAD = 10240
ROWS_PER_TILE = N_NODES_PAD // NS
ZROWS = 128

_f32 = jnp.float32


def _node_proj_body(x_ref, ws_ref, wt_ref, xs_ref, xt_ref):
    x = x_ref[...]
    xs_ref[...] = jnp.dot(x, ws_ref[...], preferred_element_type=_f32)
    xt_ref[...] = jnp.dot(x, wt_ref[...], preferred_element_type=_f32)


def _edge_proj_body(attr_ref, we_ref, be_ref, ea_ref):
    ea_ref[...] = (
        jnp.dot(attr_ref[...], we_ref[...], preferred_element_type=_f32)
        + be_ref[...]
    )


def _sc_edge_body(xs_hbm, xt_hbm, ea_hbm, row_hbm, col_hbm,
                  acc_hbm, deg_hbm,
                  row0, col0, row1, col1, row2, col2, row3, col3,
                  rowt, colt,
                  ea0, xt0, h0, ea1, xt1, h1,
                  deg_v, tmp_a, tmp_b, acc_sh,
                  sx0, st0, se0, ss0, sx1, st1, se1, ss1,
                  sri0, sci0, sri1, sci1, sri2, sci2, sri3, sci3):
    c = lax.axis_index("c")
    s = lax.axis_index("s")
    wid = c * NS + s

    zvec = jnp.zeros((16,), _f32)

    def dzero(i, _):
        deg_v[pl.ds(i * 16, 16)] = zvec
        return 0

    lax.fori_loop(0, N_NODES_PAD // 16, dzero, 0)

    def zrow(i, _):
        for j in range(NODE_DIM // 16):
            xt0[i, pl.ds(j * 16, 16)] = zvec
        return 0

    lax.fori_loop(0, CHUNK, zrow, 0)

    def zcopy(i, _):
        pltpu.sync_copy(
            xt0, acc_sh.at[pl.ds(s * ROWS_PER_TILE + i * CHUNK, CHUNK)]
        )
        return 0

    lax.fori_loop(0, ROWS_PER_TILE // CHUNK, zcopy, 0)

    pltpu.sync_copy(
        xt0.at[pl.ds(0, 16)],
        acc_sh.at[pl.ds(s * ROWS_PER_TILE + (ROWS_PER_TILE // CHUNK) * CHUNK, 16)],
    )

    plsc.subcore_barrier()

    lane = lax.broadcasted_iota(jnp.int32, (16,), 0)

    def count_degrees(idx):
        srt, _ = plsc.sort_key_val(idx, idx)
        tmp_a[pl.ds(0, 16)] = srt
        nxt = plsc.load_gather(tmp_a, [jnp.minimum(lane + 1, 15)])
        is_last = jnp.logical_or(srt != nxt, lane == 15)
        cm = plsc.cummax(jnp.where(is_last, lane, -1))
        tmp_b[pl.ds(0, 16)] = cm
        prev = plsc.load_gather(tmp_b, [jnp.maximum(lane - 1, 0)])
        prev = jnp.where(lane == 0, -1, prev)
        cnt = (lane - prev).astype(_f32)
        plsc.addupdate_scatter(deg_v, [srt], cnt, mask=is_last)

    def relu_rows(ea_v, xt_v, h_v, npairs):
        def pairbody(p, _):
            r0 = 2 * p
            r1 = 2 * p + 1
            for j in range(NODE_DIM // 16):
                sl = pl.ds(j * 16, 16)
                h_v[r0, sl] = jnp.maximum(
                    h_v[r0, sl] + xt_v[r0, sl] + ea_v[r0, sl], 0.0)
                h_v[r1, sl] = jnp.maximum(
                    h_v[r1, sl] + xt_v[r1, sl] + ea_v[r1, sl], 0.0)
            return 0

        lax.fori_loop(0, npairs, pairbody, 0)

    base_t = wid * E_PER_W + N_CHUNKS * CHUNK
    pltpu.sync_copy(row_hbm.at[pl.ds(base_t, TAIL)], rowt)
    pltpu.sync_copy(col_hbm.at[pl.ds(base_t, TAIL)], colt)
    g1 = pltpu.async_copy(xs_hbm.at[rowt], h0.at[pl.ds(0, TAIL)], sx0)
    g2 = pltpu.async_copy(xt_hbm.at[colt], xt0.at[pl.ds(0, TAIL)], st0)
    pltpu.sync_copy(ea_hbm.at[pl.ds(base_t, TAIL)], ea0.at[pl.ds(0, TAIL)])
    g1.wait()
    g2.wait()
    relu_rows(ea0, xt0, h0, TAIL // 2)
    count_degrees(colt[pl.ds(0, 16)])
    pltpu.sync_copy(h0.at[pl.ds(0, TAIL)], acc_sh.at[colt], add=True)

    ibufs = ((row0, col0, sri0, sci0), (row1, col1, sri1, sci1),
             (row2, col2, sri2, sci2), (row3, col3, sri3, sci3))
    dbufs = ((ea0, xt0, h0, sx0, st0, se0, ss0),
             (ea1, xt1, h1, sx1, st1, se1, ss1))

    def ibase(t):
        return wid * E_PER_W + t * CHUNK

    def idx_load_async(t, islot):
        (r, cl, sr, sc_) = ibufs[islot]
        base = ibase(t)
        pltpu.async_copy(row_hbm.at[pl.ds(base, CHUNK)], r, sr)
        pltpu.async_copy(col_hbm.at[pl.ds(base, CHUNK)], cl, sc_)

    def prefetch_data(d, t, islot):
        (ea_v, xt_v, h_v, se_x, se_t, se_e, _) = dbufs[d]
        (r, cl, sr, sc_) = ibufs[islot]
        base = ibase(t)
        pltpu.make_async_copy(row_hbm.at[pl.ds(base, CHUNK)], r, sr).wait()
        pltpu.make_async_copy(col_hbm.at[pl.ds(base, CHUNK)], cl, sc_).wait()
        pltpu.async_copy(xs_hbm.at[r], h_v, se_x)
        pltpu.async_copy(xt_hbm.at[cl], xt_v, se_t)
        pltpu.async_copy(ea_hbm.at[pl.ds(base, CHUNK)], ea_v, se_e)

    def wait_in(d, t, islot):
        (ea_v, xt_v, h_v, se_x, se_t, se_e, _) = dbufs[d]
        (r, cl, _, _) = ibufs[islot]
        base = ibase(t)
        pltpu.make_async_copy(xs_hbm.at[r], h_v, se_x).wait()
        pltpu.make_async_copy(xt_hbm.at[cl], xt_v, se_t).wait()
        pltpu.make_async_copy(ea_hbm.at[pl.ds(base, CHUNK)], ea_v, se_e).wait()

    def scatter_issue(d, islot):
        (_, _, h_v, _, _, _, se_s) = dbufs[d]
        cl = ibufs[islot][1]
        pltpu.async_copy(h_v, acc_sh.at[cl], se_s, add=True)

    def scatter_drain(d, islot):
        (_, _, h_v, _, _, _, se_s) = dbufs[d]
        cl = ibufs[islot][1]
        pltpu.make_async_copy(h_v, acc_sh.at[cl], se_s).wait()

    def count_chunk(islot):
        cl = ibufs[islot][1]
        for g in range(CHUNK // 16):
            count_degrees(cl[pl.ds(g * 16, 16)])

    def process(d, t, islot):
        (ea_v, xt_v, h_v, _, _, _, _) = dbufs[d]
        relu_rows(ea_v, xt_v, h_v, CHUNK // 2)
        scatter_issue(d, islot)
        count_chunk(islot)

    for j in range(4):
        idx_load_async(j, j)
    prefetch_data(0, 0, 0)
    prefetch_data(1, 1, 1)

    def quad(k, _):
        t0 = 4 * k
        wait_in(0, t0, 0)
        process(0, t0, 0)
        wait_in(1, t0 + 1, 1)
        scatter_drain(0, 0)
        idx_load_async(t0 + 4, 0)
        prefetch_data(0, t0 + 2, 2)
        process(1, t0 + 1, 1)
        scatter_drain(1, 1)
        idx_load_async(t0 + 5, 1)
        prefetch_data(1, t0 + 3, 3)
        wait_in(0, t0 + 2, 2)
        process(0, t0 + 2, 2)
        wait_in(1, t0 + 3, 3)
        scatter_drain(0, 2)
        idx_load_async(t0 + 6, 2)
        prefetch_data(0, t0 + 4, 0)
        process(1, t0 + 3, 3)
        scatter_drain(1, 3)
        idx_load_async(t0 + 7, 3)
        prefetch_data(1, t0 + 5, 1)
        return 0

    lax.fori_loop(0, (N_CHUNKS - 4) // 4, quad, 0)

    t0 = N_CHUNKS - 4
    wait_in(0, t0, 0)
    process(0, t0, 0)
    wait_in(1, t0 + 1, 1)
    scatter_drain(0, 0)
    prefetch_data(0, t0 + 2, 2)
    process(1, t0 + 1, 1)
    scatter_drain(1, 1)
    prefetch_data(1, t0 + 3, 3)
    wait_in(0, t0 + 2, 2)
    process(0, t0 + 2, 2)
    wait_in(1, t0 + 3, 3)
    process(1, t0 + 3, 3)
    scatter_drain(0, 2)
    scatter_drain(1, 3)

    plsc.subcore_barrier()

    r0 = s * ROWS_PER_TILE
    pltpu.sync_copy(
        acc_sh.at[pl.ds(r0, ROWS_PER_TILE)],
        acc_hbm.at[pl.ds(c * N_NODES_PAD + r0, ROWS_PER_TILE)],
    )
    pltpu.sync_copy(deg_v, deg_hbm.at[wid])


def _phase_b(xs, xt, ea, row, col):
    mesh = plsc.VectorSubcoreMesh(core_axis_name="c", subcore_axis_name="s")
    idx_t = lambda n: pltpu.VMEM((n,), jnp.int32)
    buf_t = lambda n: pltpu.VMEM((n, NODE_DIM), _f32)
    return pl.kernel(
        _sc_edge_body,
        out_type=(
            jax.ShapeDtypeStruct((NC * N_NODES_PAD, NODE_DIM), _f32),
            jax.ShapeDtypeStruct((NW, N_NODES_PAD), _f32),
        ),
        mesh=mesh,
        compiler_params=pltpu.CompilerParams(needs_layout_passes=False),
        scratch_types=[
            idx_t(CHUNK), idx_t(CHUNK), idx_t(CHUNK), idx_t(CHUNK),
            idx_t(CHUNK), idx_t(CHUNK), idx_t(CHUNK), idx_t(CHUNK),
            idx_t(TAIL), idx_t(TAIL),
            buf_t(CHUNK), buf_t(CHUNK), buf_t(CHUNK),
            buf_t(CHUNK), buf_t(CHUNK), buf_t(CHUNK),
            pltpu.VMEM((N_NODES_PAD,), _f32),
            pltpu.VMEM((16,), jnp.int32),
            pltpu.VMEM((16,), jnp.int32),
            pltpu.VMEM_SHARED((N_NODES_PAD, NODE_DIM), _f32),
        ] + [pltpu.SemaphoreType.DMA] * 16,
    )(xs, xt, ea, row, col)


def _node_mlp_body(agg0_ref, agg1_ref, degt_ref, x_ref, we2_ref, be2_ref,
                   wn1a_ref, wn1x_ref, bn1_ref, wn2_ref, bn2_ref, out_ref):
    aggsum = agg0_ref[...] + agg1_ref[...]
    deg = jnp.sum(degt_ref[...], axis=1, keepdims=True)
    aggregated = (
        jnp.dot(aggsum, we2_ref[...], preferred_element_type=_f32)
        + deg * be2_ref[...]
    )
    x = x_ref[...]
    h2 = jnp.maximum(
        jnp.dot(x, wn1x_ref[...], preferred_element_type=_f32)
        + jnp.dot(aggregated, wn1a_ref[...], preferred_element_type=_f32)
        + bn1_ref[...],
        0.0,
    )
    out_ref[...] = jnp.maximum(
        jnp.dot(h2, wn2_ref[...], preferred_element_type=_f32)
        + bn2_ref[...] + x,
        0.0,
    )


def _phase_c(agg0, agg1, degt, x, we2, be2, wn1a, wn1x, bn1, wn2, bn2):
    def full(r, c):
        return pl.BlockSpec((r, c), lambda i: (0, 0))

    return pl.pallas_call(
        _node_mlp_body,
        grid=(5,),
        in_specs=[
            pl.BlockSpec((2000, NODE_DIM), lambda i: (i, 0)),
            pl.BlockSpec((2000, NODE_DIM), lambda i: (i, 0)),
            pl.BlockSpec((2000, NW), lambda i: (i, 0)),
            pl.BlockSpec((2000, NODE_DIM), lambda i: (i, 0)),
            full(NODE_DIM, NODE_DIM),
            full(1, NODE_DIM),
            full(NODE_DIM, NODE_DIM),
            full(NODE_DIM, NODE_DIM),
            full(1, NODE_DIM),
            full(NODE_DIM, NODE_DIM),
            full(1, NODE_DIM),
        ],
        out_specs=pl.BlockSpec((2000, NODE_DIM), lambda i: (i, 0)),
        out_shape=jax.ShapeDtypeStruct((N_NODES, NODE_DIM), _f32),
    )(agg0, agg1, degt, x, we2, be2, wn1a, wn1x, bn1, wn2, bn2)


def kernel(x, edge_index, edge_attr, We1, be1, We2, be2, Wn1, bn1, Wn2, bn2):
    row = edge_index[0].astype(jnp.int32)
    col = edge_index[1].astype(jnp.int32)

    ws = We1[:NODE_DIM]
    wt = We1[NODE_DIM:2 * NODE_DIM]
    we = We1[2 * NODE_DIM:]

    xs, xt = pl.pallas_call(
        _node_proj_body,
        grid=(5,),
        in_specs=[
            pl.BlockSpec((2000, NODE_DIM), lambda i: (i, 0)),
            pl.BlockSpec((NODE_DIM, NODE_DIM), lambda i: (0, 0)),
            pl.BlockSpec((NODE_DIM, NODE_DIM), lambda i: (0, 0)),
        ],
        out_specs=[
            pl.BlockSpec((2000, NODE_DIM), lambda i: (i, 0)),
            pl.BlockSpec((2000, NODE_DIM), lambda i: (i, 0)),
        ],
        out_shape=[jax.ShapeDtypeStruct((N_NODES, NODE_DIM), _f32)] * 2,
    )(x, ws, wt)

    ea = pl.pallas_call(
        _edge_proj_body,
        grid=(20,),
        in_specs=[
            pl.BlockSpec((16000, EDGE_DIM), lambda i: (i, 0)),
            pl.BlockSpec((EDGE_DIM, NODE_DIM), lambda i: (0, 0)),
            pl.BlockSpec((1, NODE_DIM), lambda i: (0, 0)),
        ],
        out_specs=pl.BlockSpec((16000, NODE_DIM), lambda i: (i, 0)),
        out_shape=jax.ShapeDtypeStruct((N_EDGES, NODE_DIM), _f32),
    )(edge_attr, we, be1.reshape(1, NODE_DIM))

    acc, deg = _phase_b(xs, xt, ea, row, col)

    degt = deg.T[:N_NODES]
    return _phase_c(
        acc[:N_NODES], acc[N_NODES_PAD:N_NODES_PAD + N_NODES], degt, x,
        We2, be2.reshape(1, NODE_DIM),
        Wn1[NODE_DIM:], Wn1[:NODE_DIM],
        bn1.reshape(1, NODE_DIM), Wn2, bn2.reshape(1, NODE_DIM),
    )
